# baseline (speedup 1.0000x reference)
import functools

import jax
import jax.numpy as jnp
from jax import lax
from jax.experimental import pallas as pl
from jax.experimental.pallas import tpu as pltpu
from jax.experimental.pallas import tpu_sc as plsc

_N = 10000
_E = 320000
_D_IN = 128
_HID = 128
_CLS = 64

_NC = 2
_NS = 16
_L = 16
_NW = _NC * _NS
_EPW = _E // _NW
_B = 64
_EPP = 10560
_NB = _EPP // _B
_NPAD = 10240
_RPT = _NPAD // _NS


def _sc_mesh():
    return plsc.VectorSubcoreMesh(
        core_axis_name="c", subcore_axis_name="s", num_cores=_NC, num_subcores=_NS
    )


_K = 3
_NCHUNK = _NB // _K


def _deg_body(dst_hbm, out_hbm, dst_all, ones_v, zeros_v, deg_sh, isem, sem):
    c = lax.axis_index("c")
    s = lax.axis_index("s")
    wid = s * _NC + c

    pltpu.async_copy(dst_hbm.at[wid], dst_all, isem)

    for off in sorted({min(i * _L, _B - _L) for i in range((_B + _L - 1) // _L)}):
        ones_v[pl.ds(off, _L)] = jnp.full((_L,), 1.0, jnp.float32)

    def fill_zeros(i, _):
        zeros_v[pl.ds(i * _L, _L)] = jnp.zeros((_L,), jnp.float32)
        return 0

    lax.fori_loop(0, _RPT // _L, fill_zeros, 0)

    pltpu.sync_copy(zeros_v, deg_sh.at[pl.ds(s * _RPT, _RPT)])
    pltpu.make_async_copy(dst_hbm.at[wid], dst_all, isem).wait()
    plsc.subcore_barrier()

    def step(chunk, _):
        base = chunk * _K
        hs = [
            pltpu.async_copy(ones_v, deg_sh.at[dst_all.at[base + i]], sem, add=True)
            for i in range(_K)
        ]
        for h in hs:
            h.wait()
        return 0

    lax.fori_loop(0, _NCHUNK, step, 0)
    plsc.subcore_barrier()
    pltpu.sync_copy(deg_sh.at[pl.ds(s * _RPT, _RPT)], out_hbm.at[c, pl.ds(s * _RPT, _RPT)])


def _make_deg():
    return pl.kernel(
        _deg_body,
        out_type=jax.ShapeDtypeStruct((_NC, _NPAD), jnp.float32),
        mesh=_sc_mesh(),
        scratch_types=[
            pltpu.VMEM((_NB, _B), jnp.int32),
            pltpu.VMEM((_B,), jnp.float32),
            pltpu.VMEM((_RPT,), jnp.float32),
            pltpu.VMEM_SHARED((_NPAD,), jnp.float32),
            pltpu.SemaphoreType.DMA,
            pltpu.SemaphoreType.DMA,
        ],
    )


def _prop_body(
    t_hbm, src_hbm, dst_hbm, out_hbm, src_all, dst_all, rows_v, acc_sh, isem, gsems, ssems, *, d
):
    c = lax.axis_index("c")
    s = lax.axis_index("s")
    wid = s * _NC + c

    pltpu.async_copy(src_hbm.at[wid], src_all, isem)
    pltpu.async_copy(dst_hbm.at[wid], dst_all, isem)

    nvec = d // _L

    def fill_zero(i, _):
        k = i // (_B * nvec)
        r = (i // nvec) % _B
        q = i % nvec
        rows_v[k, r, pl.ds(q * _L, _L)] = jnp.zeros((_L,), jnp.float32)
        return 0

    lax.fori_loop(0, _K * _B * nvec, fill_zero, 0)

    for j in range(_RPT // _B):
        pltpu.sync_copy(rows_v.at[0], acc_sh.at[pl.ds(s * _RPT + j * _B, _B)])
    pltpu.make_async_copy(src_hbm.at[wid], src_all, isem).wait()
    pltpu.make_async_copy(dst_hbm.at[wid], dst_all, isem).wait()
    plsc.subcore_barrier()

    def step(chunk, _):
        base = chunk * _K
        ghs = [
            pltpu.async_copy(t_hbm.at[src_all.at[base + i]], rows_v.at[i], gsems[i])
            for i in range(_K)
        ]
        shs = []
        for i in range(_K):
            ghs[i].wait()
            shs.append(
                pltpu.async_copy(
                    rows_v.at[i], acc_sh.at[dst_all.at[base + i]], ssems[i], add=True
                )
            )
        for h in shs:
            h.wait()
        return 0

    lax.fori_loop(0, _NCHUNK, step, 0)
    plsc.subcore_barrier()
    pltpu.sync_copy(
        acc_sh.at[pl.ds(s * _RPT, _RPT)], out_hbm.at[c, pl.ds(s * _RPT, _RPT)]
    )


def _make_prop(d):
    return pl.kernel(
        functools.partial(_prop_body, d=d),
        out_type=jax.ShapeDtypeStruct((_NC, _NPAD, d), jnp.float32),
        mesh=_sc_mesh(),
        scratch_types=[
            pltpu.VMEM((_NB, _B), jnp.int32),
            pltpu.VMEM((_NB, _B), jnp.int32),
            pltpu.VMEM((_K, _B, d), jnp.float32),
            pltpu.VMEM_SHARED((_NPAD, d), jnp.float32),
            pltpu.SemaphoreType.DMA,
            [pltpu.SemaphoreType.DMA] * _K,
            [pltpu.SemaphoreType.DMA] * _K,
        ],
        compiler_params=pltpu.CompilerParams(use_tc_tiling_on_sc=False),
    )


_R = 1000


def _norm_from(deg_blk):
    deg = deg_blk[:, 0:1] + deg_blk[:, 1:2]
    return lax.rsqrt(jnp.maximum(deg, 1.0))


def _scale_body(degT_ref, x_ref, o_ref):
    o_ref[...] = x_ref[...] * _norm_from(degT_ref[...])


def _mid_body(s1p_ref, degT_ref, w1_ref, b1_ref, w2_ref, o_ref):
    norm = _norm_from(degT_ref[...])
    s1 = (s1p_ref[0] + s1p_ref[1]) * norm
    h = jnp.dot(s1, w1_ref[...], preferred_element_type=jnp.float32) + b1_ref[...]
    h = jnp.where(h > 0.0, h, jnp.exp(h) - 1.0)
    o_ref[...] = jnp.dot(h, w2_ref[...], preferred_element_type=jnp.float32) * norm


def _final_body(s2p_ref, degT_ref, b2_ref, o_ref):
    norm = _norm_from(degT_ref[...])
    o_ref[...] = (s2p_ref[0] + s2p_ref[1]) * norm + b2_ref[...]


def _scale_call(degT, x):
    return pl.pallas_call(
        _scale_body,
        grid=(_N // _R,),
        in_specs=[
            pl.BlockSpec((_R, 2), lambda i: (i, 0)),
            pl.BlockSpec((_R, _D_IN), lambda i: (i, 0)),
        ],
        out_specs=pl.BlockSpec((_R, _D_IN), lambda i: (i, 0)),
        out_shape=jax.ShapeDtypeStruct((_N, _D_IN), jnp.float32),
    )(degT, x)


def _mid_call(s1p, degT, w1, b1, w2):
    return pl.pallas_call(
        _mid_body,
        grid=(_N // _R,),
        in_specs=[
            pl.BlockSpec((_NC, _R, _HID), lambda i: (0, i, 0)),
            pl.BlockSpec((_R, 2), lambda i: (i, 0)),
            pl.BlockSpec((_D_IN, _HID), lambda i: (0, 0)),
            pl.BlockSpec((1, _HID), lambda i: (0, 0)),
            pl.BlockSpec((_HID, _CLS), lambda i: (0, 0)),
        ],
        out_specs=pl.BlockSpec((_R, _CLS), lambda i: (i, 0)),
        out_shape=jax.ShapeDtypeStruct((_N, _CLS), jnp.float32),
    )(s1p, degT, w1, b1, w2)


def _final_call(s2p, degT, b2):
    return pl.pallas_call(
        _final_body,
        grid=(_N // _R,),
        in_specs=[
            pl.BlockSpec((_NC, _R, _CLS), lambda i: (0, i, 0)),
            pl.BlockSpec((_R, 2), lambda i: (i, 0)),
            pl.BlockSpec((1, _CLS), lambda i: (0, 0)),
        ],
        out_specs=pl.BlockSpec((_R, _CLS), lambda i: (i, 0)),
        out_shape=jax.ShapeDtypeStruct((_N, _CLS), jnp.float32),
    )(s2p, degT, b2)


def kernel(features, edge_index, W1, b1, W2, b2):
    srcw = edge_index[0].reshape(_NW, _EPW)
    dstw = edge_index[1].reshape(_NW, _EPW)
    pad = _EPP - _EPW
    src = jnp.concatenate(
        [srcw, jnp.zeros((_NW, pad), jnp.int32)], axis=1).reshape(_NW, _NB, _B)
    dst = jnp.concatenate(
        [dstw, jnp.full((_NW, pad), _N, jnp.int32)], axis=1).reshape(_NW, _NB, _B)

    deg2 = _make_deg()(dst)
    degT = deg2.T

    t1 = _scale_call(degT, features)
    s1p = _make_prop(_D_IN)(t1, src, dst)
    t2 = _mid_call(s1p, degT, W1, b1.reshape(1, _HID), W2)
    s2p = _make_prop(_CLS)(t2, src, dst)
    return _final_call(s2p, degT, b2.reshape(1, _CLS))


# speedup vs baseline: 4.0258x; 4.0258x over previous
import functools

import jax
import jax.numpy as jnp
from jax import lax
from jax.experimental import pallas as pl
from jax.experimental.pallas import tpu as pltpu
from jax.experimental.pallas import tpu_sc as plsc

_N = 10000
_E = 320000
_D_IN = 128
_HID = 128
_CLS = 64

_NC = 2
_NS = 16
_L = 16
_NW = _NC * _NS
_EPW = _E // _NW
_B = 40
_EPP = 10000
_NB = _EPP // _B
_NPAD = 10240
_RPT = _NPAD // _NS


def _sc_mesh():
    return plsc.VectorSubcoreMesh(
        core_axis_name="c", subcore_axis_name="s", num_cores=_NC, num_subcores=_NS
    )


_K = 5
_NCHUNK = _NB // _K


def _deg_body(dst_hbm, out_hbm, dst_all, ones_v, zeros_v, deg_sh, isem, sem):
    c = lax.axis_index("c")
    s = lax.axis_index("s")
    wid = s * _NC + c

    pltpu.async_copy(dst_hbm.at[wid], dst_all, isem)

    for off in sorted({min(i * _L, _B - _L) for i in range((_B + _L - 1) // _L)}):
        ones_v[pl.ds(off, _L)] = jnp.full((_L,), 1.0, jnp.float32)

    def fill_zeros(i, _):
        zeros_v[pl.ds(i * _L, _L)] = jnp.zeros((_L,), jnp.float32)
        return 0

    lax.fori_loop(0, _RPT // _L, fill_zeros, 0)

    pltpu.sync_copy(zeros_v, deg_sh.at[pl.ds(s * _RPT, _RPT)])
    pltpu.make_async_copy(dst_hbm.at[wid], dst_all, isem).wait()
    plsc.subcore_barrier()

    def step(chunk, _):
        base = chunk * _K
        hs = [
            pltpu.async_copy(ones_v, deg_sh.at[dst_all.at[base + i]], sem, add=True)
            for i in range(_K)
        ]
        for h in hs:
            h.wait()
        return 0

    lax.fori_loop(0, _NCHUNK, step, 0)
    plsc.subcore_barrier()
    pltpu.sync_copy(deg_sh.at[pl.ds(s * _RPT, _RPT)], out_hbm.at[c, pl.ds(s * _RPT, _RPT)])


def _make_deg():
    return pl.kernel(
        _deg_body,
        out_type=jax.ShapeDtypeStruct((_NC, _NPAD), jnp.float32),
        mesh=_sc_mesh(),
        scratch_types=[
            pltpu.VMEM((_NB, _B), jnp.int32),
            pltpu.VMEM((_B,), jnp.float32),
            pltpu.VMEM((_RPT,), jnp.float32),
            pltpu.VMEM_SHARED((_NPAD,), jnp.float32),
            pltpu.SemaphoreType.DMA,
            pltpu.SemaphoreType.DMA,
        ],
    )


def _prop_body(
    t_hbm, src_hbm, dst_hbm, out_hbm, src_all, dst_all, rows_v, acc_sh, isem, gsems, ssems, *, d
):
    c = lax.axis_index("c")
    s = lax.axis_index("s")
    wid = s * _NC + c

    pltpu.async_copy(src_hbm.at[wid], src_all, isem)
    pltpu.async_copy(dst_hbm.at[wid], dst_all, isem)

    nvec = d // _L

    def fill_zero(i, _):
        k = i // (_B * nvec)
        r = (i // nvec) % _B
        q = i % nvec
        rows_v[k, r, pl.ds(q * _L, _L)] = jnp.zeros((_L,), jnp.float32)
        return 0

    lax.fori_loop(0, _K * _B * nvec, fill_zero, 0)

    for j in range(_RPT // _B):
        pltpu.sync_copy(rows_v.at[0], acc_sh.at[pl.ds(s * _RPT + j * _B, _B)])
    pltpu.make_async_copy(src_hbm.at[wid], src_all, isem).wait()
    pltpu.make_async_copy(dst_hbm.at[wid], dst_all, isem).wait()
    plsc.subcore_barrier()

    def step(chunk, _):
        base = chunk * _K
        ghs = [
            pltpu.async_copy(t_hbm.at[src_all.at[base + i]], rows_v.at[i], gsems[i])
            for i in range(_K)
        ]
        shs = []
        for i in range(_K):
            ghs[i].wait()
            shs.append(
                pltpu.async_copy(
                    rows_v.at[i], acc_sh.at[dst_all.at[base + i]], ssems[i], add=True
                )
            )
        for h in shs:
            h.wait()
        return 0

    lax.fori_loop(0, _NCHUNK, step, 0)
    plsc.subcore_barrier()
    pltpu.sync_copy(
        acc_sh.at[pl.ds(s * _RPT, _RPT)], out_hbm.at[c, pl.ds(s * _RPT, _RPT)]
    )


def _make_prop(d):
    return pl.kernel(
        functools.partial(_prop_body, d=d),
        out_type=jax.ShapeDtypeStruct((_NC, _NPAD, d), jnp.float32),
        mesh=_sc_mesh(),
        scratch_types=[
            pltpu.VMEM((_NB, _B), jnp.int32),
            pltpu.VMEM((_NB, _B), jnp.int32),
            pltpu.VMEM((_K, _B, d), jnp.float32),
            pltpu.VMEM_SHARED((_NPAD, d), jnp.float32),
            pltpu.SemaphoreType.DMA,
            [pltpu.SemaphoreType.DMA] * _K,
            [pltpu.SemaphoreType.DMA] * _K,
        ],
        compiler_params=pltpu.CompilerParams(use_tc_tiling_on_sc=False),
    )


_R = 1000


def _norm_from(deg_blk):
    deg = deg_blk[:, 0:1] + deg_blk[:, 1:2]
    return lax.rsqrt(jnp.maximum(deg, 1.0))


def _scale_body(degT_ref, x_ref, o_ref):
    o_ref[...] = x_ref[...] * _norm_from(degT_ref[...])


def _mid_body(s1p_ref, degT_ref, w1_ref, b1_ref, w2_ref, o_ref):
    norm = _norm_from(degT_ref[...])
    s1 = (s1p_ref[0] + s1p_ref[1]) * norm
    h = jnp.dot(s1, w1_ref[...], preferred_element_type=jnp.float32) + b1_ref[...]
    h = jnp.where(h > 0.0, h, jnp.exp(h) - 1.0)
    o_ref[...] = jnp.dot(h, w2_ref[...], preferred_element_type=jnp.float32) * norm


def _final_body(s2p_ref, degT_ref, b2_ref, o_ref):
    norm = _norm_from(degT_ref[...])
    o_ref[...] = (s2p_ref[0] + s2p_ref[1]) * norm + b2_ref[...]


def _scale_call(degT, x):
    return pl.pallas_call(
        _scale_body,
        grid=(_N // _R,),
        in_specs=[
            pl.BlockSpec((_R, 2), lambda i: (i, 0)),
            pl.BlockSpec((_R, _D_IN), lambda i: (i, 0)),
        ],
        out_specs=pl.BlockSpec((_R, _D_IN), lambda i: (i, 0)),
        out_shape=jax.ShapeDtypeStruct((_N, _D_IN), jnp.float32),
    )(degT, x)


def _mid_call(s1p, degT, w1, b1, w2):
    return pl.pallas_call(
        _mid_body,
        grid=(_N // _R,),
        in_specs=[
            pl.BlockSpec((_NC, _R, _HID), lambda i: (0, i, 0)),
            pl.BlockSpec((_R, 2), lambda i: (i, 0)),
            pl.BlockSpec((_D_IN, _HID), lambda i: (0, 0)),
            pl.BlockSpec((1, _HID), lambda i: (0, 0)),
            pl.BlockSpec((_HID, _CLS), lambda i: (0, 0)),
        ],
        out_specs=pl.BlockSpec((_R, _CLS), lambda i: (i, 0)),
        out_shape=jax.ShapeDtypeStruct((_N, _CLS), jnp.float32),
    )(s1p, degT, w1, b1, w2)


def _final_call(s2p, degT, b2):
    return pl.pallas_call(
        _final_body,
        grid=(_N // _R,),
        in_specs=[
            pl.BlockSpec((_NC, _R, _CLS), lambda i: (0, i, 0)),
            pl.BlockSpec((_R, 2), lambda i: (i, 0)),
            pl.BlockSpec((1, _CLS), lambda i: (0, 0)),
        ],
        out_specs=pl.BlockSpec((_R, _CLS), lambda i: (i, 0)),
        out_shape=jax.ShapeDtypeStruct((_N, _CLS), jnp.float32),
    )(s2p, degT, b2)


def kernel(features, edge_index, W1, b1, W2, b2):
    srcw = edge_index[0].reshape(_NW, _EPW)
    dstw = edge_index[1].reshape(_NW, _EPW)
    pad = _EPP - _EPW
    src = jnp.concatenate(
        [srcw, jnp.zeros((_NW, pad), jnp.int32)], axis=1).reshape(_NW, _NB, _B)
    dst = jnp.concatenate(
        [dstw, jnp.full((_NW, pad), _N, jnp.int32)], axis=1).reshape(_NW, _NB, _B)

    deg2 = _make_deg()(dst)
    degT = deg2.T

    t1 = _scale_call(degT, features)
    s1p = _make_prop(_D_IN)(t1, src, dst)
    t2 = _mid_call(s1p, degT, W1, b1.reshape(1, _HID), W2)
    s2p = _make_prop(_CLS)(t2, src, dst)
    return _final_call(s2p, degT, b2.reshape(1, _CLS))


# speedup vs baseline: 4.1765x; 1.0375x over previous
import functools

import jax
import jax.numpy as jnp
from jax import lax
from jax.experimental import pallas as pl
from jax.experimental.pallas import tpu as pltpu
from jax.experimental.pallas import tpu_sc as plsc

_N = 10000
_E = 320000
_D_IN = 128
_HID = 128
_CLS = 64

_NC = 2
_NS = 16
_L = 16
_NW = _NC * _NS
_EPW = _E // _NW
_BP = {128: 40, 64: 80}
_BDEG = 80
_NPAD = 10240
_RPT = _NPAD // _NS


def _sc_mesh():
    return plsc.VectorSubcoreMesh(
        core_axis_name="c", subcore_axis_name="s", num_cores=_NC, num_subcores=_NS
    )


_K = 5


def _deg_body(dst_hbm, out_hbm, dst_all, ones_v, zeros_v, deg_sh, isem, sem, *, b):
    c = lax.axis_index("c")
    s = lax.axis_index("s")
    wid = s * _NC + c

    pltpu.async_copy(dst_hbm.at[wid], dst_all, isem)

    for off in sorted({min(i * _L, b - _L) for i in range((b + _L - 1) // _L)}):
        ones_v[pl.ds(off, _L)] = jnp.full((_L,), 1.0, jnp.float32)

    def fill_zeros(i, _):
        zeros_v[pl.ds(i * _L, _L)] = jnp.zeros((_L,), jnp.float32)
        return 0

    lax.fori_loop(0, _RPT // _L, fill_zeros, 0)

    pltpu.sync_copy(zeros_v, deg_sh.at[pl.ds(s * _RPT, _RPT)])
    pltpu.make_async_copy(dst_hbm.at[wid], dst_all, isem).wait()
    plsc.subcore_barrier()

    def step(chunk, _):
        base = chunk * _K
        hs = [
            pltpu.async_copy(ones_v, deg_sh.at[dst_all.at[base + i]], sem, add=True)
            for i in range(_K)
        ]
        for h in hs:
            h.wait()
        return 0

    lax.fori_loop(0, (_EPW // b) // _K, step, 0)
    plsc.subcore_barrier()
    pltpu.sync_copy(deg_sh.at[pl.ds(s * _RPT, _RPT)], out_hbm.at[c, pl.ds(s * _RPT, _RPT)])


def _make_deg(b=_BDEG):
    return pl.kernel(
        functools.partial(_deg_body, b=b),
        out_type=jax.ShapeDtypeStruct((_NC, _NPAD), jnp.float32),
        mesh=_sc_mesh(),
        scratch_types=[
            pltpu.VMEM((_EPW // b, b), jnp.int32),
            pltpu.VMEM((b,), jnp.float32),
            pltpu.VMEM((_RPT,), jnp.float32),
            pltpu.VMEM_SHARED((_NPAD,), jnp.float32),
            pltpu.SemaphoreType.DMA,
            pltpu.SemaphoreType.DMA,
        ],
    )


def _prop_body(
    t_hbm, src_hbm, dst_hbm, out_hbm, src_all, dst_all, rows_v, acc_sh, isem, gsems, ssems, *, d, b
):
    c = lax.axis_index("c")
    s = lax.axis_index("s")
    wid = s * _NC + c

    pltpu.async_copy(src_hbm.at[wid], src_all, isem)
    pltpu.async_copy(dst_hbm.at[wid], dst_all, isem)

    nvec = d // _L

    def fill_zero(i, _):
        k = i // (b * nvec)
        r = (i // nvec) % b
        q = i % nvec
        rows_v[k, r, pl.ds(q * _L, _L)] = jnp.zeros((_L,), jnp.float32)
        return 0

    lax.fori_loop(0, _K * b * nvec, fill_zero, 0)

    zoffs = sorted({min(j * b, _RPT - b) for j in range((_RPT + b - 1) // b)})
    for zo in zoffs:
        pltpu.sync_copy(rows_v.at[0], acc_sh.at[pl.ds(s * _RPT + zo, b)])
    pltpu.make_async_copy(src_hbm.at[wid], src_all, isem).wait()
    pltpu.make_async_copy(dst_hbm.at[wid], dst_all, isem).wait()
    plsc.subcore_barrier()

    def step(chunk, _):
        base = chunk * _K
        ghs = [
            pltpu.async_copy(t_hbm.at[src_all.at[base + i]], rows_v.at[i], gsems[i])
            for i in range(_K)
        ]
        shs = []
        for i in range(_K):
            ghs[i].wait()
            shs.append(
                pltpu.async_copy(
                    rows_v.at[i], acc_sh.at[dst_all.at[base + i]], ssems[i], add=True
                )
            )
        for h in shs:
            h.wait()
        return 0

    lax.fori_loop(0, (_EPW // b) // _K, step, 0)
    plsc.subcore_barrier()
    pltpu.sync_copy(
        acc_sh.at[pl.ds(s * _RPT, _RPT)], out_hbm.at[c, pl.ds(s * _RPT, _RPT)]
    )


def _make_prop(d):
    b = _BP[d]
    return pl.kernel(
        functools.partial(_prop_body, d=d, b=b),
        out_type=jax.ShapeDtypeStruct((_NC, _NPAD, d), jnp.float32),
        mesh=_sc_mesh(),
        scratch_types=[
            pltpu.VMEM((_EPW // b, b), jnp.int32),
            pltpu.VMEM((_EPW // b, b), jnp.int32),
            pltpu.VMEM((_K, b, d), jnp.float32),
            pltpu.VMEM_SHARED((_NPAD, d), jnp.float32),
            pltpu.SemaphoreType.DMA,
            [pltpu.SemaphoreType.DMA] * _K,
            [pltpu.SemaphoreType.DMA] * _K,
        ],
        compiler_params=pltpu.CompilerParams(use_tc_tiling_on_sc=False),
    )


_R = 1000


def _norm_from(deg_blk):
    deg = deg_blk[:, 0:1] + deg_blk[:, 1:2]
    return lax.rsqrt(jnp.maximum(deg, 1.0))


def _scale_body(degT_ref, x_ref, o_ref):
    o_ref[...] = x_ref[...] * _norm_from(degT_ref[...])


def _mid_body(s1p_ref, degT_ref, w1_ref, b1_ref, w2_ref, o_ref):
    norm = _norm_from(degT_ref[...])
    s1 = (s1p_ref[0] + s1p_ref[1]) * norm
    h = jnp.dot(s1, w1_ref[...], preferred_element_type=jnp.float32) + b1_ref[...]
    h = jnp.where(h > 0.0, h, jnp.exp(h) - 1.0)
    o_ref[...] = jnp.dot(h, w2_ref[...], preferred_element_type=jnp.float32) * norm


def _final_body(s2p_ref, degT_ref, b2_ref, o_ref):
    norm = _norm_from(degT_ref[...])
    o_ref[...] = (s2p_ref[0] + s2p_ref[1]) * norm + b2_ref[...]


def _scale_call(degT, x):
    return pl.pallas_call(
        _scale_body,
        grid=(_N // _R,),
        in_specs=[
            pl.BlockSpec((_R, 2), lambda i: (i, 0)),
            pl.BlockSpec((_R, _D_IN), lambda i: (i, 0)),
        ],
        out_specs=pl.BlockSpec((_R, _D_IN), lambda i: (i, 0)),
        out_shape=jax.ShapeDtypeStruct((_N, _D_IN), jnp.float32),
    )(degT, x)


def _mid_call(s1p, degT, w1, b1, w2):
    return pl.pallas_call(
        _mid_body,
        grid=(_N // _R,),
        in_specs=[
            pl.BlockSpec((_NC, _R, _HID), lambda i: (0, i, 0)),
            pl.BlockSpec((_R, 2), lambda i: (i, 0)),
            pl.BlockSpec((_D_IN, _HID), lambda i: (0, 0)),
            pl.BlockSpec((1, _HID), lambda i: (0, 0)),
            pl.BlockSpec((_HID, _CLS), lambda i: (0, 0)),
        ],
        out_specs=pl.BlockSpec((_R, _CLS), lambda i: (i, 0)),
        out_shape=jax.ShapeDtypeStruct((_N, _CLS), jnp.float32),
    )(s1p, degT, w1, b1, w2)


def _final_call(s2p, degT, b2):
    return pl.pallas_call(
        _final_body,
        grid=(_N // _R,),
        in_specs=[
            pl.BlockSpec((_NC, _R, _CLS), lambda i: (0, i, 0)),
            pl.BlockSpec((_R, 2), lambda i: (i, 0)),
            pl.BlockSpec((1, _CLS), lambda i: (0, 0)),
        ],
        out_specs=pl.BlockSpec((_R, _CLS), lambda i: (i, 0)),
        out_shape=jax.ShapeDtypeStruct((_N, _CLS), jnp.float32),
    )(s2p, degT, b2)


def kernel(features, edge_index, W1, b1, W2, b2):
    srcw = edge_index[0].reshape(_NW, _EPW)
    dstw = edge_index[1].reshape(_NW, _EPW)
    src40 = srcw.reshape(_NW, _EPW // _BP[128], _BP[128])
    dst40 = dstw.reshape(_NW, _EPW // _BP[128], _BP[128])
    src80 = srcw.reshape(_NW, _EPW // _BP[64], _BP[64])
    dst80 = dstw.reshape(_NW, _EPW // _BP[64], _BP[64])
    dstdeg = dstw.reshape(_NW, _EPW // _BDEG, _BDEG)

    deg2 = _make_deg()(dstdeg)
    degT = deg2.T

    t1 = _scale_call(degT, features)
    s1p = _make_prop(_D_IN)(t1, src40, dst40)
    t2 = _mid_call(s1p, degT, W1, b1.reshape(1, _HID), W2)
    s2p = _make_prop(_CLS)(t2, src80, dst80)
    return _final_call(s2p, degT, b2.reshape(1, _CLS))
